# baseline (speedup 1.0000x reference)
import functools

import numpy as np

import jax
import jax.numpy as jnp
from jax import lax
from jax.experimental import pallas as pl
from jax.experimental.pallas import tpu as pltpu
from jax.experimental.pallas import tpu_sc as plsc

N = 10000
E = 320000
D = 128
H = 64
DE = 16
ITERS = 3

NC = 2
NS = 16
NW = NC * NS
EPT = E // NW
CH = 125
NCH = EPT // CH
NPS = N // NS
CHS = 128
NCHS = E // CHS
NCHS_CEIL = -(-NCHS // NW)


def _scatter_perm():
    s = np.arange(E)
    c, t = s // CHS, s % CHS
    rr = c * (CHS // 8) + t // 8
    k = t % 8
    b, j = rr // (_BM // 8), rr % (_BM // 8)
    return b * _BM + k * (_BM // 8) + j

_BM = 16000
_BN = 2000

_SC_PARAMS = pltpu.CompilerParams(use_tc_tiling_on_sc=False)


@functools.cache
def _make_sc_gather():
    mesh = plsc.VectorSubcoreMesh(
        core_axis_name="c", subcore_axis_name="s", num_cores=NC, num_subcores=NS
    )

    @functools.partial(
        pl.kernel,
        mesh=mesh,
        compiler_params=_SC_PARAMS,
        out_type=jax.ShapeDtypeStruct((2, E, D), jnp.float32),
        scratch_types=[
            pltpu.VMEM((2, NCH, CH), jnp.int32),
            pltpu.VMEM((CH, D), jnp.float32),
            pltpu.VMEM((CH, D), jnp.float32),
            pltpu.VMEM((CH, D), jnp.float32),
            pltpu.VMEM((CH, D), jnp.float32),
            pltpu.SemaphoreType.DMA,
            pltpu.SemaphoreType.DMA,
            pltpu.SemaphoreType.DMA,
            pltpu.SemaphoreType.DMA,
            pltpu.SemaphoreType.DMA,
            pltpu.SemaphoreType.DMA,
            pltpu.SemaphoreType.DMA,
            pltpu.SemaphoreType.DMA,
        ],
    )
    def gather_k(
        pq_hbm, eidx_hbm, out_hbm, idx_v, bu0, bv0, bu1, bv1,
        gu0, gv0, gu1, gv1, wu0, wv0, wu1, wv1,
    ):
        wid = lax.axis_index("s") * NC + lax.axis_index("c")
        base = wid * EPT
        pltpu.sync_copy(eidx_hbm.at[wid], idx_v)

        def gath(r, j, buf, sem):
            pltpu.async_copy(pq_hbm.at[idx_v.at[r, j]], buf, sem)

        def wait_gath(r, j, buf, sem):
            pltpu.make_async_copy(pq_hbm.at[idx_v.at[r, j]], buf, sem).wait()

        def wout(r, j, buf, sem):
            pltpu.async_copy(buf, out_hbm.at[r, pl.ds(base + j * CH, CH)], sem)

        def wait_wout(r, j, buf, sem):
            pltpu.make_async_copy(
                buf, out_hbm.at[r, pl.ds(base + j * CH, CH)], sem
            ).wait()

        gath(0, 0, bu0, gu0)
        gath(1, 0, bv0, gv0)

        @pl.loop(0, NCH // 2)
        def _(i):
            c0 = 2 * i
            c1 = c0 + 1
            wait_gath(0, c0, bu0, gu0)
            wout(0, c0, bu0, wu0)
            wait_gath(1, c0, bv0, gv0)
            wout(1, c0, bv0, wv0)

            @pl.when(i > 0)
            def _():
                wait_wout(0, c1 - 2, bu1, wu1)
                wait_wout(1, c1 - 2, bv1, wv1)

            gath(0, c1, bu1, gu1)
            gath(1, c1, bv1, gv1)
            wait_gath(0, c1, bu1, gu1)
            wout(0, c1, bu1, wu1)
            wait_gath(1, c1, bv1, gv1)
            wout(1, c1, bv1, wv1)

            @pl.when(i + 1 < NCH // 2)
            def _():
                wait_wout(0, c0, bu0, wu0)
                wait_wout(1, c0, bv0, wv0)
                gath(0, c0 + 2, bu0, gu0)
                gath(1, c0 + 2, bv0, gv0)

        wait_wout(0, NCH - 2, bu0, wu0)
        wait_wout(1, NCH - 2, bv0, wv0)
        wait_wout(0, NCH - 1, bu1, wu1)
        wait_wout(1, NCH - 1, bv1, wv1)

    return gather_k


def _sc_gather(pq, eidx):
    return _make_sc_gather()(pq, eidx)


@functools.cache
def _make_sc_scatter():
    mesh = plsc.VectorSubcoreMesh(
        core_axis_name="c", subcore_axis_name="s", num_cores=NC, num_subcores=NS
    )

    @functools.partial(
        pl.kernel,
        mesh=mesh,
        compiler_params=_SC_PARAMS,
        out_type=jax.ShapeDtypeStruct((2, N, DE), jnp.float32),
        scratch_types=[
            pltpu.VMEM((2, CHS), jnp.int32),
            pltpu.VMEM((CHS // 8, 128), jnp.float32),
            pltpu.VMEM((CHS, DE), jnp.float32),
            pltpu.VMEM_SHARED((N, DE), jnp.float32),
        ],
    )
    def scatter_k(m_hbm, sidx_hbm, zero_hbm, out_hbm, idx_v, m16_v, m_v, acc_sh):
        cid = lax.axis_index("c")
        sid = lax.axis_index("s")
        wid = sid * NC + cid
        pltpu.sync_copy(
            zero_hbm.at[pl.ds(sid * NPS, NPS)], acc_sh.at[pl.ds(sid * NPS, NPS)]
        )
        plsc.subcore_barrier()

        @pl.loop(0, NCHS_CEIL)
        def _(j):
            c = j * NW + wid

            @pl.when(c < NCHS)
            def _():
                pltpu.sync_copy(sidx_hbm.at[c], idx_v)
                for r in range(2):
                    pltpu.sync_copy(
                        m_hbm.at[r, pl.ds(c * (CHS // 8), CHS // 8)], m16_v
                    )

                    @pl.loop(0, CHS // 8)
                    def _(i):
                        for k in range(8):
                            m_v[8 * i + k, :] = m16_v[i, pl.ds(16 * k, 16)]

                    pltpu.sync_copy(m_v, acc_sh.at[idx_v.at[r]], add=True)

        plsc.subcore_barrier()
        pltpu.sync_copy(
            acc_sh.at[pl.ds(sid * NPS, NPS)], out_hbm.at[cid, pl.ds(sid * NPS, NPS)]
        )

    return scatter_k


def _sc_scatter(m, sidx, zero_a):
    return _make_sc_scatter()(m, sidx, zero_a)


def _proj_body(h_ref, w1at_ref, w1bt_ref, b1_ref, pq_ref):
    hblk = h_ref[...]
    p = jnp.dot(hblk, w1at_ref[...], preferred_element_type=jnp.float32)
    q = jnp.dot(hblk, w1bt_ref[...], preferred_element_type=jnp.float32)
    pq_ref[...] = jnp.concatenate([p, q + b1_ref[...]], axis=1)


def _tc_proj(h, w1at, w1bt, b1r):
    return pl.pallas_call(
        _proj_body,
        grid=(N // _BN,),
        in_specs=[
            pl.BlockSpec((_BN, D), lambda i: (i, 0)),
            pl.BlockSpec((D, H), lambda i: (0, 0)),
            pl.BlockSpec((D, H), lambda i: (0, 0)),
            pl.BlockSpec((1, H), lambda i: (0, 0)),
        ],
        out_specs=pl.BlockSpec((_BN, D), lambda i: (i, 0)),
        out_shape=jax.ShapeDtypeStruct((N, D), jnp.float32),
    )(h, w1at, w1bt, b1r)


def _mlp_body(ruv_ref, w2_ref, b2_ref, w3_ref, b3_ref, m_ref):
    ru = ruv_ref[0]
    rv = ruv_ref[1]
    s = jnp.concatenate([ru[:, :H] + rv[:, H:], rv[:, :H] + ru[:, H:]], axis=0)
    m1 = jnp.maximum(s, 0.0)
    m2 = jnp.dot(m1, w2_ref[...], preferred_element_type=jnp.float32) + b2_ref[...]
    m2 = jnp.maximum(m2, 0.0)
    m3 = jnp.dot(m2, w3_ref[...], preferred_element_type=jnp.float32) + b3_ref[...]
    pk = _BM // 8
    for r in range(2):
        mr = m3[r * _BM : (r + 1) * _BM]
        packed = jnp.concatenate(
            [mr[k * pk : (k + 1) * pk] for k in range(8)], axis=1
        )
        m_ref[r] = packed


def _tc_mlp(ruv, w2t, b2r, w3t, b3r):
    return pl.pallas_call(
        _mlp_body,
        grid=(E // _BM,),
        in_specs=[
            pl.BlockSpec((2, _BM, D), lambda i: (0, i, 0)),
            pl.BlockSpec((H, H), lambda i: (0, 0)),
            pl.BlockSpec((1, H), lambda i: (0, 0)),
            pl.BlockSpec((H, DE), lambda i: (0, 0)),
            pl.BlockSpec((1, DE), lambda i: (0, 0)),
        ],
        out_specs=pl.BlockSpec((2, _BM // 8, 128), lambda i: (0, i, 0)),
        out_shape=jax.ShapeDtypeStruct((2, E // 8, 128), jnp.float32),
    )(ruv, w2t, b2r, w3t, b3r)


def _gru_body(ap_ref, h_ref, wiht_ref, whht_ref, bih_ref, bhh_ref, ho_ref):
    a = ap_ref[0] + ap_ref[1]
    hblk = h_ref[...]
    gi = jnp.dot(a, wiht_ref[...], preferred_element_type=jnp.float32) + bih_ref[...]
    gh = jnp.dot(hblk, whht_ref[...], preferred_element_type=jnp.float32) + bhh_ref[...]
    r = jax.nn.sigmoid(gi[:, :D] + gh[:, :D])
    z = jax.nn.sigmoid(gi[:, D : 2 * D] + gh[:, D : 2 * D])
    n = jnp.tanh(gi[:, 2 * D :] + r * gh[:, 2 * D :])
    ho_ref[...] = (1.0 - z) * n + z * hblk


def _tc_gru(parts, h, wiht, whht, bihr, bhhr):
    return pl.pallas_call(
        _gru_body,
        grid=(N // _BN,),
        in_specs=[
            pl.BlockSpec((2, _BN, DE), lambda i: (0, i, 0)),
            pl.BlockSpec((_BN, D), lambda i: (i, 0)),
            pl.BlockSpec((DE, 3 * D), lambda i: (0, 0)),
            pl.BlockSpec((D, 3 * D), lambda i: (0, 0)),
            pl.BlockSpec((1, 3 * D), lambda i: (0, 0)),
            pl.BlockSpec((1, 3 * D), lambda i: (0, 0)),
        ],
        out_specs=pl.BlockSpec((_BN, D), lambda i: (i, 0)),
        out_shape=jax.ShapeDtypeStruct((N, D), jnp.float32),
    )(parts, h, wiht, whht, bihr, bhhr)


def kernel(node_features, edges, W1, b1, W2, b2, W3, b3, W_ih, W_hh, b_ih, b_hh):
    eidx = edges.reshape(2, NW, NCH, CH).transpose(1, 0, 2, 3)
    sidx = edges[:, _scatter_perm()].reshape(2, NCHS, CHS).transpose(1, 0, 2)
    w1at = W1[:, :D].T
    w1bt = W1[:, D:].T
    b1r = b1.reshape(1, H)
    w2t = W2.T
    b2r = b2.reshape(1, H)
    w3t = W3.T
    b3r = b3.reshape(1, DE)
    wiht = W_ih.T
    whht = W_hh.T
    bihr = b_ih.reshape(1, 3 * D)
    bhhr = b_hh.reshape(1, 3 * D)
    zero_a = jnp.zeros((N, DE), jnp.float32)

    h = node_features
    for _ in range(ITERS):
        pq = _tc_proj(h, w1at, w1bt, b1r)
        ruv = _sc_gather(pq, eidx)
        m = _tc_mlp(ruv, w2t, b2r, w3t, b3r)
        parts = _sc_scatter(m, sidx, zero_a)
        h = _tc_gru(parts, h, wiht, whht, bihr, bhhr)
    return h


# speedup vs baseline: 1.0017x; 1.0017x over previous
import functools

import numpy as np

import jax
import jax.numpy as jnp
from jax import lax
from jax.experimental import pallas as pl
from jax.experimental.pallas import tpu as pltpu
from jax.experimental.pallas import tpu_sc as plsc

N = 10000
E = 320000
D = 128
H = 64
DE = 16
ITERS = 3

NC = 2
NS = 16
NW = NC * NS
EPT = E // NW
CH = 125
NCH = EPT // CH
NPS = N // NS
CHS = 128
NCHS = E // CHS
NCHS_CEIL = -(-NCHS // NW)


def _scatter_perm():
    s = np.arange(E)
    c, t = s // CHS, s % CHS
    rr = c * (CHS // 8) + t // 8
    k = t % 8
    b, j = rr // (_BM // 8), rr % (_BM // 8)
    return b * _BM + k * (_BM // 8) + j

_BM = 8000
_BN = 2000

_SC_PARAMS = pltpu.CompilerParams(use_tc_tiling_on_sc=False)


@functools.cache
def _make_sc_gather():
    mesh = plsc.VectorSubcoreMesh(
        core_axis_name="c", subcore_axis_name="s", num_cores=NC, num_subcores=NS
    )

    @functools.partial(
        pl.kernel,
        mesh=mesh,
        compiler_params=_SC_PARAMS,
        out_type=jax.ShapeDtypeStruct((2, E, D), jnp.float32),
        scratch_types=[
            pltpu.VMEM((2, NCH, CH), jnp.int32),
            pltpu.VMEM((CH, D), jnp.float32),
            pltpu.VMEM((CH, D), jnp.float32),
            pltpu.VMEM((CH, D), jnp.float32),
            pltpu.VMEM((CH, D), jnp.float32),
            pltpu.SemaphoreType.DMA,
            pltpu.SemaphoreType.DMA,
            pltpu.SemaphoreType.DMA,
            pltpu.SemaphoreType.DMA,
            pltpu.SemaphoreType.DMA,
            pltpu.SemaphoreType.DMA,
            pltpu.SemaphoreType.DMA,
            pltpu.SemaphoreType.DMA,
        ],
    )
    def gather_k(
        pq_hbm, eidx_hbm, out_hbm, idx_v, bu0, bv0, bu1, bv1,
        gu0, gv0, gu1, gv1, wu0, wv0, wu1, wv1,
    ):
        wid = lax.axis_index("s") * NC + lax.axis_index("c")
        base = wid * EPT
        pltpu.sync_copy(eidx_hbm.at[wid], idx_v)

        def gath(r, j, buf, sem):
            pltpu.async_copy(pq_hbm.at[idx_v.at[r, j]], buf, sem)

        def wait_gath(r, j, buf, sem):
            pltpu.make_async_copy(pq_hbm.at[idx_v.at[r, j]], buf, sem).wait()

        def wout(r, j, buf, sem):
            pltpu.async_copy(buf, out_hbm.at[r, pl.ds(base + j * CH, CH)], sem)

        def wait_wout(r, j, buf, sem):
            pltpu.make_async_copy(
                buf, out_hbm.at[r, pl.ds(base + j * CH, CH)], sem
            ).wait()

        gath(0, 0, bu0, gu0)
        gath(1, 0, bv0, gv0)

        @pl.loop(0, NCH // 2)
        def _(i):
            c0 = 2 * i
            c1 = c0 + 1
            wait_gath(0, c0, bu0, gu0)
            wout(0, c0, bu0, wu0)
            wait_gath(1, c0, bv0, gv0)
            wout(1, c0, bv0, wv0)

            @pl.when(i > 0)
            def _():
                wait_wout(0, c1 - 2, bu1, wu1)
                wait_wout(1, c1 - 2, bv1, wv1)

            gath(0, c1, bu1, gu1)
            gath(1, c1, bv1, gv1)
            wait_gath(0, c1, bu1, gu1)
            wout(0, c1, bu1, wu1)
            wait_gath(1, c1, bv1, gv1)
            wout(1, c1, bv1, wv1)

            @pl.when(i + 1 < NCH // 2)
            def _():
                wait_wout(0, c0, bu0, wu0)
                wait_wout(1, c0, bv0, wv0)
                gath(0, c0 + 2, bu0, gu0)
                gath(1, c0 + 2, bv0, gv0)

        wait_wout(0, NCH - 2, bu0, wu0)
        wait_wout(1, NCH - 2, bv0, wv0)
        wait_wout(0, NCH - 1, bu1, wu1)
        wait_wout(1, NCH - 1, bv1, wv1)

    return gather_k


def _sc_gather(pq, eidx):
    return _make_sc_gather()(pq, eidx)


@functools.cache
def _make_sc_scatter():
    mesh = plsc.VectorSubcoreMesh(
        core_axis_name="c", subcore_axis_name="s", num_cores=NC, num_subcores=NS
    )

    @functools.partial(
        pl.kernel,
        mesh=mesh,
        compiler_params=_SC_PARAMS,
        out_type=jax.ShapeDtypeStruct((2, N, DE), jnp.float32),
        scratch_types=[
            pltpu.VMEM((2, CHS), jnp.int32),
            pltpu.VMEM((CHS // 8, 128), jnp.float32),
            pltpu.VMEM((CHS, DE), jnp.float32),
            pltpu.VMEM_SHARED((N, DE), jnp.float32),
        ],
    )
    def scatter_k(m_hbm, sidx_hbm, zero_hbm, out_hbm, idx_v, m16_v, m_v, acc_sh):
        cid = lax.axis_index("c")
        sid = lax.axis_index("s")
        wid = sid * NC + cid
        pltpu.sync_copy(
            zero_hbm.at[pl.ds(sid * NPS, NPS)], acc_sh.at[pl.ds(sid * NPS, NPS)]
        )
        plsc.subcore_barrier()

        @pl.loop(0, NCHS_CEIL)
        def _(j):
            c = j * NW + wid

            @pl.when(c < NCHS)
            def _():
                pltpu.sync_copy(sidx_hbm.at[c], idx_v)
                for r in range(2):
                    pltpu.sync_copy(
                        m_hbm.at[r, pl.ds(c * (CHS // 8), CHS // 8)], m16_v
                    )

                    @pl.loop(0, CHS // 8)
                    def _(i):
                        for k in range(8):
                            m_v[8 * i + k, :] = m16_v[i, pl.ds(16 * k, 16)]

                    pltpu.sync_copy(m_v, acc_sh.at[idx_v.at[r]], add=True)

        plsc.subcore_barrier()
        pltpu.sync_copy(
            acc_sh.at[pl.ds(sid * NPS, NPS)], out_hbm.at[cid, pl.ds(sid * NPS, NPS)]
        )

    return scatter_k


def _sc_scatter(m, sidx, zero_a):
    return _make_sc_scatter()(m, sidx, zero_a)


def _proj_body(h_ref, w1at_ref, w1bt_ref, b1_ref, pq_ref):
    hblk = h_ref[...]
    p = jnp.dot(hblk, w1at_ref[...], preferred_element_type=jnp.float32)
    q = jnp.dot(hblk, w1bt_ref[...], preferred_element_type=jnp.float32)
    pq_ref[...] = jnp.concatenate([p, q + b1_ref[...]], axis=1)


def _tc_proj(h, w1at, w1bt, b1r):
    return pl.pallas_call(
        _proj_body,
        grid=(N // _BN,),
        in_specs=[
            pl.BlockSpec((_BN, D), lambda i: (i, 0)),
            pl.BlockSpec((D, H), lambda i: (0, 0)),
            pl.BlockSpec((D, H), lambda i: (0, 0)),
            pl.BlockSpec((1, H), lambda i: (0, 0)),
        ],
        out_specs=pl.BlockSpec((_BN, D), lambda i: (i, 0)),
        out_shape=jax.ShapeDtypeStruct((N, D), jnp.float32),
    )(h, w1at, w1bt, b1r)


def _mlp_body(ruv_ref, w2_ref, b2_ref, w3_ref, b3_ref, m_ref):
    ru = ruv_ref[0]
    rv = ruv_ref[1]
    s = jnp.concatenate([ru[:, :H] + rv[:, H:], rv[:, :H] + ru[:, H:]], axis=0)
    m1 = jnp.maximum(s, 0.0)
    m2 = jnp.dot(m1, w2_ref[...], preferred_element_type=jnp.float32) + b2_ref[...]
    m2 = jnp.maximum(m2, 0.0)
    m3 = jnp.dot(m2, w3_ref[...], preferred_element_type=jnp.float32) + b3_ref[...]
    pk = _BM // 8
    for r in range(2):
        mr = m3[r * _BM : (r + 1) * _BM]
        packed = jnp.concatenate(
            [mr[k * pk : (k + 1) * pk] for k in range(8)], axis=1
        )
        m_ref[r] = packed


def _tc_mlp(ruv, w2t, b2r, w3t, b3r):
    return pl.pallas_call(
        _mlp_body,
        grid=(E // _BM,),
        in_specs=[
            pl.BlockSpec((2, _BM, D), lambda i: (0, i, 0)),
            pl.BlockSpec((H, H), lambda i: (0, 0)),
            pl.BlockSpec((1, H), lambda i: (0, 0)),
            pl.BlockSpec((H, DE), lambda i: (0, 0)),
            pl.BlockSpec((1, DE), lambda i: (0, 0)),
        ],
        out_specs=pl.BlockSpec((2, _BM // 8, 128), lambda i: (0, i, 0)),
        out_shape=jax.ShapeDtypeStruct((2, E // 8, 128), jnp.float32),
    )(ruv, w2t, b2r, w3t, b3r)


def _gru_body(ap_ref, h_ref, wiht_ref, whht_ref, bih_ref, bhh_ref, ho_ref):
    a = ap_ref[0] + ap_ref[1]
    hblk = h_ref[...]
    gi = jnp.dot(a, wiht_ref[...], preferred_element_type=jnp.float32) + bih_ref[...]
    gh = jnp.dot(hblk, whht_ref[...], preferred_element_type=jnp.float32) + bhh_ref[...]
    r = jax.nn.sigmoid(gi[:, :D] + gh[:, :D])
    z = jax.nn.sigmoid(gi[:, D : 2 * D] + gh[:, D : 2 * D])
    n = jnp.tanh(gi[:, 2 * D :] + r * gh[:, 2 * D :])
    ho_ref[...] = (1.0 - z) * n + z * hblk


def _tc_gru(parts, h, wiht, whht, bihr, bhhr):
    return pl.pallas_call(
        _gru_body,
        grid=(N // _BN,),
        in_specs=[
            pl.BlockSpec((2, _BN, DE), lambda i: (0, i, 0)),
            pl.BlockSpec((_BN, D), lambda i: (i, 0)),
            pl.BlockSpec((DE, 3 * D), lambda i: (0, 0)),
            pl.BlockSpec((D, 3 * D), lambda i: (0, 0)),
            pl.BlockSpec((1, 3 * D), lambda i: (0, 0)),
            pl.BlockSpec((1, 3 * D), lambda i: (0, 0)),
        ],
        out_specs=pl.BlockSpec((_BN, D), lambda i: (i, 0)),
        out_shape=jax.ShapeDtypeStruct((N, D), jnp.float32),
    )(parts, h, wiht, whht, bihr, bhhr)


def kernel(node_features, edges, W1, b1, W2, b2, W3, b3, W_ih, W_hh, b_ih, b_hh):
    eidx = edges.reshape(2, NW, NCH, CH).transpose(1, 0, 2, 3)
    sidx = edges[:, _scatter_perm()].reshape(2, NCHS, CHS).transpose(1, 0, 2)
    w1at = W1[:, :D].T
    w1bt = W1[:, D:].T
    b1r = b1.reshape(1, H)
    w2t = W2.T
    b2r = b2.reshape(1, H)
    w3t = W3.T
    b3r = b3.reshape(1, DE)
    wiht = W_ih.T
    whht = W_hh.T
    bihr = b_ih.reshape(1, 3 * D)
    bhhr = b_hh.reshape(1, 3 * D)
    zero_a = jnp.zeros((N, DE), jnp.float32)

    h = node_features
    for _ in range(ITERS):
        pq = _tc_proj(h, w1at, w1bt, b1r)
        ruv = _sc_gather(pq, eidx)
        m = _tc_mlp(ruv, w2t, b2r, w3t, b3r)
        parts = _sc_scatter(m, sidx, zero_a)
        h = _tc_gru(parts, h, wiht, whht, bihr, bhhr)
    return h


# speedup vs baseline: 1.0145x; 1.0127x over previous
import functools

import numpy as np

import jax
import jax.numpy as jnp
from jax import lax
from jax.experimental import pallas as pl
from jax.experimental.pallas import tpu as pltpu
from jax.experimental.pallas import tpu_sc as plsc

N = 10000
E = 320000
D = 128
H = 64
DE = 16
ITERS = 3

NC = 2
NS = 16
NW = NC * NS
EH = E // 2
CH = 125
NPS = N // NS
CHS = 128


def _scatter_perm(ne):
    s = np.arange(ne)
    c, t = s // CHS, s % CHS
    rr = c * (CHS // 8) + t // 8
    k = t % 8
    b, j = rr // (_BM // 8), rr % (_BM // 8)
    return b * _BM + k * (_BM // 8) + j

_BM = 8000
_BN = 2000

_SC_PARAMS = pltpu.CompilerParams(use_tc_tiling_on_sc=False)


@functools.cache
def _make_sc_gather(ne):
    ept = ne // NW
    nch = ept // CH
    mesh = plsc.VectorSubcoreMesh(
        core_axis_name="c", subcore_axis_name="s", num_cores=NC, num_subcores=NS
    )

    @functools.partial(
        pl.kernel,
        mesh=mesh,
        compiler_params=_SC_PARAMS,
        out_type=jax.ShapeDtypeStruct((2, ne, D), jnp.float32),
        scratch_types=[
            pltpu.VMEM((2, nch, CH), jnp.int32),
            pltpu.VMEM((CH, D), jnp.float32),
            pltpu.VMEM((CH, D), jnp.float32),
            pltpu.VMEM((CH, D), jnp.float32),
            pltpu.VMEM((CH, D), jnp.float32),
            pltpu.SemaphoreType.DMA,
            pltpu.SemaphoreType.DMA,
            pltpu.SemaphoreType.DMA,
            pltpu.SemaphoreType.DMA,
            pltpu.SemaphoreType.DMA,
            pltpu.SemaphoreType.DMA,
            pltpu.SemaphoreType.DMA,
            pltpu.SemaphoreType.DMA,
        ],
    )
    def gather_k(
        pq_hbm, eidx_hbm, out_hbm, idx_v, bu0, bv0, bu1, bv1,
        gu0, gv0, gu1, gv1, wu0, wv0, wu1, wv1,
    ):
        wid = lax.axis_index("s") * NC + lax.axis_index("c")
        base = wid * ept
        pltpu.sync_copy(eidx_hbm.at[wid], idx_v)

        def gath(r, j, buf, sem):
            pltpu.async_copy(pq_hbm.at[idx_v.at[r, j]], buf, sem)

        def wait_gath(r, j, buf, sem):
            pltpu.make_async_copy(pq_hbm.at[idx_v.at[r, j]], buf, sem).wait()

        def wout(r, j, buf, sem):
            pltpu.async_copy(buf, out_hbm.at[r, pl.ds(base + j * CH, CH)], sem)

        def wait_wout(r, j, buf, sem):
            pltpu.make_async_copy(
                buf, out_hbm.at[r, pl.ds(base + j * CH, CH)], sem
            ).wait()

        gath(0, 0, bu0, gu0)
        gath(1, 0, bv0, gv0)

        @pl.loop(0, nch // 2)
        def _(i):
            c0 = 2 * i
            c1 = c0 + 1
            wait_gath(0, c0, bu0, gu0)
            wout(0, c0, bu0, wu0)
            wait_gath(1, c0, bv0, gv0)
            wout(1, c0, bv0, wv0)

            @pl.when(i > 0)
            def _():
                wait_wout(0, c1 - 2, bu1, wu1)
                wait_wout(1, c1 - 2, bv1, wv1)

            gath(0, c1, bu1, gu1)
            gath(1, c1, bv1, gv1)
            wait_gath(0, c1, bu1, gu1)
            wout(0, c1, bu1, wu1)
            wait_gath(1, c1, bv1, gv1)
            wout(1, c1, bv1, wv1)

            @pl.when(i + 1 < nch // 2)
            def _():
                wait_wout(0, c0, bu0, wu0)
                wait_wout(1, c0, bv0, wv0)
                gath(0, c0 + 2, bu0, gu0)
                gath(1, c0 + 2, bv0, gv0)

        wait_wout(0, nch - 2, bu0, wu0)
        wait_wout(1, nch - 2, bv0, wv0)
        wait_wout(0, nch - 1, bu1, wu1)
        wait_wout(1, nch - 1, bv1, wv1)

    return gather_k


def _sc_gather(pq, eidx):
    ne = eidx.shape[0] * eidx.shape[2] * eidx.shape[3]
    return _make_sc_gather(ne)(pq, eidx)


@functools.cache
def _make_sc_scatter(ne):
    nchs = ne // CHS
    nchs_ceil = -(-nchs // NW)
    mesh = plsc.VectorSubcoreMesh(
        core_axis_name="c", subcore_axis_name="s", num_cores=NC, num_subcores=NS
    )

    @functools.partial(
        pl.kernel,
        mesh=mesh,
        compiler_params=_SC_PARAMS,
        out_type=jax.ShapeDtypeStruct((2, N, DE), jnp.float32),
        scratch_types=[
            pltpu.VMEM((2, CHS), jnp.int32),
            pltpu.VMEM((CHS // 8, 128), jnp.float32),
            pltpu.VMEM((CHS, DE), jnp.float32),
            pltpu.VMEM_SHARED((N, DE), jnp.float32),
        ],
    )
    def scatter_k(m_hbm, sidx_hbm, zero_hbm, out_hbm, idx_v, m16_v, m_v, acc_sh):
        cid = lax.axis_index("c")
        sid = lax.axis_index("s")
        wid = sid * NC + cid
        pltpu.sync_copy(
            zero_hbm.at[pl.ds(sid * NPS, NPS)], acc_sh.at[pl.ds(sid * NPS, NPS)]
        )
        plsc.subcore_barrier()

        @pl.loop(0, nchs_ceil)
        def _(j):
            c = j * NW + wid

            @pl.when(c < nchs)
            def _():
                pltpu.sync_copy(sidx_hbm.at[c], idx_v)
                for r in range(2):
                    pltpu.sync_copy(
                        m_hbm.at[r, pl.ds(c * (CHS // 8), CHS // 8)], m16_v
                    )

                    @pl.loop(0, CHS // 8)
                    def _(i):
                        for k in range(8):
                            m_v[8 * i + k, :] = m16_v[i, pl.ds(16 * k, 16)]

                    pltpu.sync_copy(m_v, acc_sh.at[idx_v.at[r]], add=True)

        plsc.subcore_barrier()
        pltpu.sync_copy(
            acc_sh.at[pl.ds(sid * NPS, NPS)], out_hbm.at[cid, pl.ds(sid * NPS, NPS)]
        )

    return scatter_k


def _sc_scatter(m, sidx, zero_a):
    return _make_sc_scatter(m.shape[1] * 8)(m, sidx, zero_a)


def _proj_body(h_ref, w1at_ref, w1bt_ref, b1_ref, pq_ref):
    hblk = h_ref[...]
    p = jnp.dot(hblk, w1at_ref[...], preferred_element_type=jnp.float32)
    q = jnp.dot(hblk, w1bt_ref[...], preferred_element_type=jnp.float32)
    pq_ref[...] = jnp.concatenate([p, q + b1_ref[...]], axis=1)


def _tc_proj(h, w1at, w1bt, b1r):
    return pl.pallas_call(
        _proj_body,
        grid=(N // _BN,),
        in_specs=[
            pl.BlockSpec((_BN, D), lambda i: (i, 0)),
            pl.BlockSpec((D, H), lambda i: (0, 0)),
            pl.BlockSpec((D, H), lambda i: (0, 0)),
            pl.BlockSpec((1, H), lambda i: (0, 0)),
        ],
        out_specs=pl.BlockSpec((_BN, D), lambda i: (i, 0)),
        out_shape=jax.ShapeDtypeStruct((N, D), jnp.float32),
    )(h, w1at, w1bt, b1r)


def _mlp_body(ruv_ref, w2_ref, b2_ref, w3_ref, b3_ref, m_ref):
    ru = ruv_ref[0]
    rv = ruv_ref[1]
    s = jnp.concatenate([ru[:, :H] + rv[:, H:], rv[:, :H] + ru[:, H:]], axis=0)
    m1 = jnp.maximum(s, 0.0)
    m2 = jnp.dot(m1, w2_ref[...], preferred_element_type=jnp.float32) + b2_ref[...]
    m2 = jnp.maximum(m2, 0.0)
    m3 = jnp.dot(m2, w3_ref[...], preferred_element_type=jnp.float32) + b3_ref[...]
    pk = _BM // 8
    for r in range(2):
        mr = m3[r * _BM : (r + 1) * _BM]
        packed = jnp.concatenate(
            [mr[k * pk : (k + 1) * pk] for k in range(8)], axis=1
        )
        m_ref[r] = packed


def _tc_mlp(ruv, w2t, b2r, w3t, b3r):
    ne = ruv.shape[1]
    return pl.pallas_call(
        _mlp_body,
        grid=(ne // _BM,),
        in_specs=[
            pl.BlockSpec((2, _BM, D), lambda i: (0, i, 0)),
            pl.BlockSpec((H, H), lambda i: (0, 0)),
            pl.BlockSpec((1, H), lambda i: (0, 0)),
            pl.BlockSpec((H, DE), lambda i: (0, 0)),
            pl.BlockSpec((1, DE), lambda i: (0, 0)),
        ],
        out_specs=pl.BlockSpec((2, _BM // 8, 128), lambda i: (0, i, 0)),
        out_shape=jax.ShapeDtypeStruct((2, ne // 8, 128), jnp.float32),
    )(ruv, w2t, b2r, w3t, b3r)


def _gru_body(ap_ref, bp_ref, h_ref, wiht_ref, whht_ref, bih_ref, bhh_ref, ho_ref):
    a = (ap_ref[0] + ap_ref[1]) + (bp_ref[0] + bp_ref[1])
    hblk = h_ref[...]
    gi = jnp.dot(a, wiht_ref[...], preferred_element_type=jnp.float32) + bih_ref[...]
    gh = jnp.dot(hblk, whht_ref[...], preferred_element_type=jnp.float32) + bhh_ref[...]
    r = jax.nn.sigmoid(gi[:, :D] + gh[:, :D])
    z = jax.nn.sigmoid(gi[:, D : 2 * D] + gh[:, D : 2 * D])
    n = jnp.tanh(gi[:, 2 * D :] + r * gh[:, 2 * D :])
    ho_ref[...] = (1.0 - z) * n + z * hblk


def _tc_gru(partsA, partsB, h, wiht, whht, bihr, bhhr):
    return pl.pallas_call(
        _gru_body,
        grid=(N // _BN,),
        in_specs=[
            pl.BlockSpec((2, _BN, DE), lambda i: (0, i, 0)),
            pl.BlockSpec((2, _BN, DE), lambda i: (0, i, 0)),
            pl.BlockSpec((_BN, D), lambda i: (i, 0)),
            pl.BlockSpec((DE, 3 * D), lambda i: (0, 0)),
            pl.BlockSpec((D, 3 * D), lambda i: (0, 0)),
            pl.BlockSpec((1, 3 * D), lambda i: (0, 0)),
            pl.BlockSpec((1, 3 * D), lambda i: (0, 0)),
        ],
        out_specs=pl.BlockSpec((_BN, D), lambda i: (i, 0)),
        out_shape=jax.ShapeDtypeStruct((N, D), jnp.float32),
    )(partsA, partsB, h, wiht, whht, bihr, bhhr)


def kernel(node_features, edges, W1, b1, W2, b2, W3, b3, W_ih, W_hh, b_ih, b_hh):
    nchh = EH // NW // CH
    edgA = edges[:, :EH]
    edgB = edges[:, EH:]
    eidxA = edgA.reshape(2, NW, nchh, CH).transpose(1, 0, 2, 3)
    eidxB = edgB.reshape(2, NW, nchh, CH).transpose(1, 0, 2, 3)
    perm = _scatter_perm(EH)
    sidxA = edgA[:, perm].reshape(2, EH // CHS, CHS).transpose(1, 0, 2)
    sidxB = edgB[:, perm].reshape(2, EH // CHS, CHS).transpose(1, 0, 2)
    w1at = W1[:, :D].T
    w1bt = W1[:, D:].T
    b1r = b1.reshape(1, H)
    w2t = W2.T
    b2r = b2.reshape(1, H)
    w3t = W3.T
    b3r = b3.reshape(1, DE)
    wiht = W_ih.T
    whht = W_hh.T
    bihr = b_ih.reshape(1, 3 * D)
    bhhr = b_hh.reshape(1, 3 * D)
    zero_a = jnp.zeros((N, DE), jnp.float32)

    h = node_features
    for _ in range(ITERS):
        pq = _tc_proj(h, w1at, w1bt, b1r)
        ruvA = _sc_gather(pq, eidxA)
        ruvB = _sc_gather(pq, eidxB)
        mA = _tc_mlp(ruvA, w2t, b2r, w3t, b3r)
        mB = _tc_mlp(ruvB, w2t, b2r, w3t, b3r)
        pA = _sc_scatter(mA, sidxA, zero_a)
        pB = _sc_scatter(mB, sidxB, zero_a)
        h = _tc_gru(pA, pB, h, wiht, whht, bihr, bhhr)
    return h


# speedup vs baseline: 1.1933x; 1.1763x over previous
import functools

import numpy as np

import jax
import jax.numpy as jnp
from jax import lax
from jax.experimental import pallas as pl
from jax.experimental.pallas import tpu as pltpu
from jax.experimental.pallas import tpu_sc as plsc

N = 10000
E = 320000
D = 128
H = 64
DE = 16
ITERS = 3

NC = 2
NS = 16
NW = NC * NS
EH = E // 2
CH = 125
NPS = N // NS
CHS = 128


def _scatter_perm(ne):
    s = np.arange(ne)
    c, t = s // CHS, s % CHS
    rr = c * (CHS // 8) + t // 8
    k = t % 8
    b, j = rr // (_BM // 8), rr % (_BM // 8)
    return b * _BM + k * (_BM // 8) + j

_BM = 8000
_BN = 2000

_SC_PARAMS = pltpu.CompilerParams(use_tc_tiling_on_sc=False)


@functools.cache
def _make_sc_gather(ne):
    ept = ne // NW
    nch = ept // CH
    mesh = plsc.VectorSubcoreMesh(
        core_axis_name="c", subcore_axis_name="s", num_cores=NC, num_subcores=NS
    )

    @functools.partial(
        pl.kernel,
        mesh=mesh,
        compiler_params=_SC_PARAMS,
        out_type=jax.ShapeDtypeStruct((2, ne, D), jnp.float32),
        scratch_types=[
            pltpu.VMEM((2, nch, CH), jnp.int32),
            pltpu.VMEM((CH, D), jnp.float32),
            pltpu.VMEM((CH, D), jnp.float32),
            pltpu.VMEM((CH, D), jnp.float32),
            pltpu.VMEM((CH, D), jnp.float32),
            pltpu.SemaphoreType.DMA,
            pltpu.SemaphoreType.DMA,
            pltpu.SemaphoreType.DMA,
            pltpu.SemaphoreType.DMA,
            pltpu.SemaphoreType.DMA,
            pltpu.SemaphoreType.DMA,
            pltpu.SemaphoreType.DMA,
            pltpu.SemaphoreType.DMA,
        ],
    )
    def gather_k(
        pq_hbm, eidx_hbm, out_hbm, idx_v, bu0, bv0, bu1, bv1,
        gu0, gv0, gu1, gv1, wu0, wv0, wu1, wv1,
    ):
        wid = lax.axis_index("s") * NC + lax.axis_index("c")
        base = wid * ept
        pltpu.sync_copy(eidx_hbm.at[wid], idx_v)

        def gath(r, j, buf, sem):
            pltpu.async_copy(pq_hbm.at[idx_v.at[r, j]], buf, sem)

        def wait_gath(r, j, buf, sem):
            pltpu.make_async_copy(pq_hbm.at[idx_v.at[r, j]], buf, sem).wait()

        def wout(r, j, buf, sem):
            pltpu.async_copy(buf, out_hbm.at[r, pl.ds(base + j * CH, CH)], sem)

        def wait_wout(r, j, buf, sem):
            pltpu.make_async_copy(
                buf, out_hbm.at[r, pl.ds(base + j * CH, CH)], sem
            ).wait()

        gath(0, 0, bu0, gu0)
        gath(1, 0, bv0, gv0)

        @pl.loop(0, nch // 2)
        def _(i):
            c0 = 2 * i
            c1 = c0 + 1
            wait_gath(0, c0, bu0, gu0)
            wout(0, c0, bu0, wu0)
            wait_gath(1, c0, bv0, gv0)
            wout(1, c0, bv0, wv0)

            @pl.when(i > 0)
            def _():
                wait_wout(0, c1 - 2, bu1, wu1)
                wait_wout(1, c1 - 2, bv1, wv1)

            gath(0, c1, bu1, gu1)
            gath(1, c1, bv1, gv1)
            wait_gath(0, c1, bu1, gu1)
            wout(0, c1, bu1, wu1)
            wait_gath(1, c1, bv1, gv1)
            wout(1, c1, bv1, wv1)

            @pl.when(i + 1 < nch // 2)
            def _():
                wait_wout(0, c0, bu0, wu0)
                wait_wout(1, c0, bv0, wv0)
                gath(0, c0 + 2, bu0, gu0)
                gath(1, c0 + 2, bv0, gv0)

        wait_wout(0, nch - 2, bu0, wu0)
        wait_wout(1, nch - 2, bv0, wv0)
        wait_wout(0, nch - 1, bu1, wu1)
        wait_wout(1, nch - 1, bv1, wv1)

    return gather_k


def _sc_gather(pq, eidx):
    ne = eidx.shape[0] * eidx.shape[2] * eidx.shape[3]
    return _make_sc_gather(ne)(pq, eidx)


@functools.cache
def _make_sc_scatter(ne):
    nchs = ne // CHS
    nchs_ceil = -(-nchs // NW)
    mesh = plsc.VectorSubcoreMesh(
        core_axis_name="c", subcore_axis_name="s", num_cores=NC, num_subcores=NS
    )

    @functools.partial(
        pl.kernel,
        mesh=mesh,
        compiler_params=_SC_PARAMS,
        out_type=jax.ShapeDtypeStruct((2, N, DE), jnp.float32),
        scratch_types=[
            pltpu.VMEM((2, CHS), jnp.int32),
            pltpu.VMEM((2, CHS), jnp.int32),
            pltpu.VMEM((2, CHS // 8, 128), jnp.float32),
            pltpu.VMEM((2, CHS // 8, 128), jnp.float32),
            pltpu.VMEM((CHS, DE), jnp.float32),
            pltpu.VMEM_SHARED((N, DE), jnp.float32),
            pltpu.SemaphoreType.DMA,
            pltpu.SemaphoreType.DMA,
        ],
    )
    def scatter_k(
        m_hbm, sidx_hbm, zero_hbm, out_hbm,
        idx0, idx1, mm0, mm1, m_v, acc_sh, s0, s1,
    ):
        cid = lax.axis_index("c")
        sid = lax.axis_index("s")
        wid = sid * NC + cid
        pltpu.sync_copy(
            zero_hbm.at[pl.ds(sid * NPS, NPS)], acc_sh.at[pl.ds(sid * NPS, NPS)]
        )
        plsc.subcore_barrier()

        def load(c, idxb, mmb, sem):
            pltpu.async_copy(sidx_hbm.at[c], idxb, sem)
            pltpu.async_copy(m_hbm.at[:, pl.ds(c * (CHS // 8), CHS // 8)], mmb, sem)

        def wait_load(c, idxb, mmb, sem):
            pltpu.make_async_copy(sidx_hbm.at[c], idxb, sem).wait()
            pltpu.make_async_copy(
                m_hbm.at[:, pl.ds(c * (CHS // 8), CHS // 8)], mmb, sem
            ).wait()

        def scat(c, idxb, mmb):
            for r in range(2):
                @pl.loop(0, CHS // 8)
                def _(i, _r=r):
                    for k in range(8):
                        m_v[8 * i + k, :] = mmb[_r, i, pl.ds(16 * k, 16)]

                pltpu.sync_copy(m_v, acc_sh.at[idxb.at[r]], add=True)

        c0_first = wid
        c1_first = NW + wid

        @pl.when(c0_first < nchs)
        def _():
            load(c0_first, idx0, mm0, s0)

        @pl.when(c1_first < nchs)
        def _():
            load(c1_first, idx1, mm1, s1)

        @pl.loop(0, -(-nchs_ceil // 2))
        def _(i):
            c0 = (2 * i) * NW + wid
            c1 = c0 + NW
            c0n = c0 + 2 * NW
            c1n = c1 + 2 * NW

            @pl.when(c0 < nchs)
            def _():
                wait_load(c0, idx0, mm0, s0)
                scat(c0, idx0, mm0)

            @pl.when(c0n < nchs)
            def _():
                load(c0n, idx0, mm0, s0)

            @pl.when(c1 < nchs)
            def _():
                wait_load(c1, idx1, mm1, s1)
                scat(c1, idx1, mm1)

            @pl.when(c1n < nchs)
            def _():
                load(c1n, idx1, mm1, s1)

        plsc.subcore_barrier()
        pltpu.sync_copy(
            acc_sh.at[pl.ds(sid * NPS, NPS)], out_hbm.at[cid, pl.ds(sid * NPS, NPS)]
        )

    return scatter_k


def _sc_scatter(m, sidx, zero_a):
    return _make_sc_scatter(m.shape[1] * 8)(m, sidx, zero_a)


def _proj_body(h_ref, w1at_ref, w1bt_ref, b1_ref, pq_ref):
    hblk = h_ref[...]
    p = jnp.dot(hblk, w1at_ref[...], preferred_element_type=jnp.float32)
    q = jnp.dot(hblk, w1bt_ref[...], preferred_element_type=jnp.float32)
    pq_ref[...] = jnp.concatenate([p, q + b1_ref[...]], axis=1)


def _tc_proj(h, w1at, w1bt, b1r):
    return pl.pallas_call(
        _proj_body,
        grid=(N // _BN,),
        in_specs=[
            pl.BlockSpec((_BN, D), lambda i: (i, 0)),
            pl.BlockSpec((D, H), lambda i: (0, 0)),
            pl.BlockSpec((D, H), lambda i: (0, 0)),
            pl.BlockSpec((1, H), lambda i: (0, 0)),
        ],
        out_specs=pl.BlockSpec((_BN, D), lambda i: (i, 0)),
        out_shape=jax.ShapeDtypeStruct((N, D), jnp.float32),
    )(h, w1at, w1bt, b1r)


def _mlp_body(ruv_ref, w2_ref, b2_ref, w3_ref, b3_ref, m_ref):
    ru = ruv_ref[0]
    rv = ruv_ref[1]
    s = jnp.concatenate([ru[:, :H] + rv[:, H:], rv[:, :H] + ru[:, H:]], axis=0)
    m1 = jnp.maximum(s, 0.0)
    m2 = jnp.dot(m1, w2_ref[...], preferred_element_type=jnp.float32) + b2_ref[...]
    m2 = jnp.maximum(m2, 0.0)
    m3 = jnp.dot(m2, w3_ref[...], preferred_element_type=jnp.float32) + b3_ref[...]
    pk = _BM // 8
    for r in range(2):
        mr = m3[r * _BM : (r + 1) * _BM]
        packed = jnp.concatenate(
            [mr[k * pk : (k + 1) * pk] for k in range(8)], axis=1
        )
        m_ref[r] = packed


def _tc_mlp(ruv, w2t, b2r, w3t, b3r):
    ne = ruv.shape[1]
    return pl.pallas_call(
        _mlp_body,
        grid=(ne // _BM,),
        in_specs=[
            pl.BlockSpec((2, _BM, D), lambda i: (0, i, 0)),
            pl.BlockSpec((H, H), lambda i: (0, 0)),
            pl.BlockSpec((1, H), lambda i: (0, 0)),
            pl.BlockSpec((H, DE), lambda i: (0, 0)),
            pl.BlockSpec((1, DE), lambda i: (0, 0)),
        ],
        out_specs=pl.BlockSpec((2, _BM // 8, 128), lambda i: (0, i, 0)),
        out_shape=jax.ShapeDtypeStruct((2, ne // 8, 128), jnp.float32),
    )(ruv, w2t, b2r, w3t, b3r)


def _gru_body(ap_ref, bp_ref, h_ref, wiht_ref, whht_ref, bih_ref, bhh_ref, ho_ref):
    a = (ap_ref[0] + ap_ref[1]) + (bp_ref[0] + bp_ref[1])
    hblk = h_ref[...]
    gi = jnp.dot(a, wiht_ref[...], preferred_element_type=jnp.float32) + bih_ref[...]
    gh = jnp.dot(hblk, whht_ref[...], preferred_element_type=jnp.float32) + bhh_ref[...]
    r = jax.nn.sigmoid(gi[:, :D] + gh[:, :D])
    z = jax.nn.sigmoid(gi[:, D : 2 * D] + gh[:, D : 2 * D])
    n = jnp.tanh(gi[:, 2 * D :] + r * gh[:, 2 * D :])
    ho_ref[...] = (1.0 - z) * n + z * hblk


def _tc_gru(partsA, partsB, h, wiht, whht, bihr, bhhr):
    return pl.pallas_call(
        _gru_body,
        grid=(N // _BN,),
        in_specs=[
            pl.BlockSpec((2, _BN, DE), lambda i: (0, i, 0)),
            pl.BlockSpec((2, _BN, DE), lambda i: (0, i, 0)),
            pl.BlockSpec((_BN, D), lambda i: (i, 0)),
            pl.BlockSpec((DE, 3 * D), lambda i: (0, 0)),
            pl.BlockSpec((D, 3 * D), lambda i: (0, 0)),
            pl.BlockSpec((1, 3 * D), lambda i: (0, 0)),
            pl.BlockSpec((1, 3 * D), lambda i: (0, 0)),
        ],
        out_specs=pl.BlockSpec((_BN, D), lambda i: (i, 0)),
        out_shape=jax.ShapeDtypeStruct((N, D), jnp.float32),
    )(partsA, partsB, h, wiht, whht, bihr, bhhr)


def kernel(node_features, edges, W1, b1, W2, b2, W3, b3, W_ih, W_hh, b_ih, b_hh):
    nchh = EH // NW // CH
    edgA = edges[:, :EH]
    edgB = edges[:, EH:]
    eidxA = edgA.reshape(2, NW, nchh, CH).transpose(1, 0, 2, 3)
    eidxB = edgB.reshape(2, NW, nchh, CH).transpose(1, 0, 2, 3)
    perm = _scatter_perm(EH)
    sidxA = edgA[:, perm].reshape(2, EH // CHS, CHS).transpose(1, 0, 2)
    sidxB = edgB[:, perm].reshape(2, EH // CHS, CHS).transpose(1, 0, 2)
    w1at = W1[:, :D].T
    w1bt = W1[:, D:].T
    b1r = b1.reshape(1, H)
    w2t = W2.T
    b2r = b2.reshape(1, H)
    w3t = W3.T
    b3r = b3.reshape(1, DE)
    wiht = W_ih.T
    whht = W_hh.T
    bihr = b_ih.reshape(1, 3 * D)
    bhhr = b_hh.reshape(1, 3 * D)
    zero_a = jnp.zeros((N, DE), jnp.float32)

    h = node_features
    for _ in range(ITERS):
        pq = _tc_proj(h, w1at, w1bt, b1r)
        ruvA = _sc_gather(pq, eidxA)
        ruvB = _sc_gather(pq, eidxB)
        mA = _tc_mlp(ruvA, w2t, b2r, w3t, b3r)
        mB = _tc_mlp(ruvB, w2t, b2r, w3t, b3r)
        pA = _sc_scatter(mA, sidxA, zero_a)
        pB = _sc_scatter(mB, sidxB, zero_a)
        h = _tc_gru(pA, pB, h, wiht, whht, bihr, bhhr)
    return h


# speedup vs baseline: 1.2046x; 1.0095x over previous
import functools

import numpy as np

import jax
import jax.numpy as jnp
from jax import lax
from jax.experimental import pallas as pl
from jax.experimental.pallas import tpu as pltpu
from jax.experimental.pallas import tpu_sc as plsc

N = 10000
E = 320000
D = 128
H = 64
DE = 16
ITERS = 3

NC = 2
NS = 16
NW = NC * NS
EH = E // 2
CH = 125
NPS = N // NS
CHS = 128


def _scatter_perm(ne):
    s = np.arange(ne)
    c, t = s // CHS, s % CHS
    rr = c * (CHS // 8) + t // 8
    k = t % 8
    b, j = rr // (_BM // 8), rr % (_BM // 8)
    return b * _BM + k * (_BM // 8) + j

_BM = 8000
_BN = 2000

_SC_PARAMS = pltpu.CompilerParams(use_tc_tiling_on_sc=False)


@functools.cache
def _make_sc_gather(ne):
    ept = ne // NW
    nch = ept // CH
    mesh = plsc.VectorSubcoreMesh(
        core_axis_name="c", subcore_axis_name="s", num_cores=NC, num_subcores=NS
    )

    @functools.partial(
        pl.kernel,
        mesh=mesh,
        compiler_params=_SC_PARAMS,
        out_type=jax.ShapeDtypeStruct((2, ne, D), jnp.float32),
        scratch_types=[
            pltpu.VMEM((2, nch, CH), jnp.int32),
            pltpu.VMEM((CH, D), jnp.float32),
            pltpu.VMEM((CH, D), jnp.float32),
            pltpu.VMEM((CH, D), jnp.float32),
            pltpu.VMEM((CH, D), jnp.float32),
            pltpu.SemaphoreType.DMA,
            pltpu.SemaphoreType.DMA,
            pltpu.SemaphoreType.DMA,
            pltpu.SemaphoreType.DMA,
            pltpu.SemaphoreType.DMA,
            pltpu.SemaphoreType.DMA,
            pltpu.SemaphoreType.DMA,
            pltpu.SemaphoreType.DMA,
        ],
    )
    def gather_k(
        pq_hbm, eidx_hbm, out_hbm, idx_v, bu0, bv0, bu1, bv1,
        gu0, gv0, gu1, gv1, wu0, wv0, wu1, wv1,
    ):
        wid = lax.axis_index("s") * NC + lax.axis_index("c")
        base = wid * ept
        pltpu.sync_copy(eidx_hbm.at[wid], idx_v)

        def gath(r, j, buf, sem):
            pltpu.async_copy(pq_hbm.at[idx_v.at[r, j]], buf, sem)

        def wait_gath(r, j, buf, sem):
            pltpu.make_async_copy(pq_hbm.at[idx_v.at[r, j]], buf, sem).wait()

        def wout(r, j, buf, sem):
            pltpu.async_copy(buf, out_hbm.at[r, pl.ds(base + j * CH, CH)], sem)

        def wait_wout(r, j, buf, sem):
            pltpu.make_async_copy(
                buf, out_hbm.at[r, pl.ds(base + j * CH, CH)], sem
            ).wait()

        gath(0, 0, bu0, gu0)
        gath(1, 0, bv0, gv0)

        @pl.loop(0, nch // 2)
        def _(i):
            c0 = 2 * i
            c1 = c0 + 1
            wait_gath(0, c0, bu0, gu0)
            wout(0, c0, bu0, wu0)
            wait_gath(1, c0, bv0, gv0)
            wout(1, c0, bv0, wv0)

            @pl.when(i > 0)
            def _():
                wait_wout(0, c1 - 2, bu1, wu1)
                wait_wout(1, c1 - 2, bv1, wv1)

            gath(0, c1, bu1, gu1)
            gath(1, c1, bv1, gv1)
            wait_gath(0, c1, bu1, gu1)
            wout(0, c1, bu1, wu1)
            wait_gath(1, c1, bv1, gv1)
            wout(1, c1, bv1, wv1)

            @pl.when(i + 1 < nch // 2)
            def _():
                wait_wout(0, c0, bu0, wu0)
                wait_wout(1, c0, bv0, wv0)
                gath(0, c0 + 2, bu0, gu0)
                gath(1, c0 + 2, bv0, gv0)

        wait_wout(0, nch - 2, bu0, wu0)
        wait_wout(1, nch - 2, bv0, wv0)
        wait_wout(0, nch - 1, bu1, wu1)
        wait_wout(1, nch - 1, bv1, wv1)

    return gather_k


def _sc_gather(pq, eidx):
    ne = eidx.shape[0] * eidx.shape[2] * eidx.shape[3]
    return _make_sc_gather(ne)(pq, eidx)


@functools.cache
def _make_sc_scatter(ne):
    nchs = ne // CHS
    nchs_ceil = -(-nchs // NW)
    mesh = plsc.VectorSubcoreMesh(
        core_axis_name="c", subcore_axis_name="s", num_cores=NC, num_subcores=NS
    )

    @functools.partial(
        pl.kernel,
        mesh=mesh,
        compiler_params=_SC_PARAMS,
        out_type=jax.ShapeDtypeStruct((2, N, DE), jnp.float32),
        scratch_types=[
            pltpu.VMEM((2, CHS), jnp.int32),
            pltpu.VMEM((2, CHS), jnp.int32),
            pltpu.VMEM((2, CHS // 8, 128), jnp.float32),
            pltpu.VMEM((2, CHS // 8, 128), jnp.float32),
            pltpu.VMEM((CHS, DE), jnp.float32),
            pltpu.VMEM_SHARED((N, DE), jnp.float32),
            pltpu.SemaphoreType.DMA,
            pltpu.SemaphoreType.DMA,
        ],
    )
    def scatter_k(
        m_hbm, sidx_hbm, zero_hbm, out_hbm,
        idx0, idx1, mm0, mm1, m_v, acc_sh, s0, s1,
    ):
        cid = lax.axis_index("c")
        sid = lax.axis_index("s")
        wid = sid * NC + cid
        pltpu.sync_copy(
            zero_hbm.at[pl.ds(sid * NPS, NPS)], acc_sh.at[pl.ds(sid * NPS, NPS)]
        )
        plsc.subcore_barrier()

        def load(c, idxb, mmb, sem):
            pltpu.async_copy(sidx_hbm.at[c], idxb, sem)
            pltpu.async_copy(m_hbm.at[:, pl.ds(c * (CHS // 8), CHS // 8)], mmb, sem)

        def wait_load(c, idxb, mmb, sem):
            pltpu.make_async_copy(sidx_hbm.at[c], idxb, sem).wait()
            pltpu.make_async_copy(
                m_hbm.at[:, pl.ds(c * (CHS // 8), CHS // 8)], mmb, sem
            ).wait()

        def scat(c, idxb, mmb):
            for r in range(2):
                @pl.loop(0, CHS // 8)
                def _(i, _r=r):
                    for k in range(8):
                        m_v[8 * i + k, :] = mmb[_r, i, pl.ds(16 * k, 16)]

                pltpu.sync_copy(m_v, acc_sh.at[idxb.at[r]], add=True)

        c0_first = wid
        c1_first = NW + wid

        @pl.when(c0_first < nchs)
        def _():
            load(c0_first, idx0, mm0, s0)

        @pl.when(c1_first < nchs)
        def _():
            load(c1_first, idx1, mm1, s1)

        @pl.loop(0, -(-nchs_ceil // 2))
        def _(i):
            c0 = (2 * i) * NW + wid
            c1 = c0 + NW
            c0n = c0 + 2 * NW
            c1n = c1 + 2 * NW

            @pl.when(c0 < nchs)
            def _():
                wait_load(c0, idx0, mm0, s0)
                scat(c0, idx0, mm0)

            @pl.when(c0n < nchs)
            def _():
                load(c0n, idx0, mm0, s0)

            @pl.when(c1 < nchs)
            def _():
                wait_load(c1, idx1, mm1, s1)
                scat(c1, idx1, mm1)

            @pl.when(c1n < nchs)
            def _():
                load(c1n, idx1, mm1, s1)

        plsc.subcore_barrier()
        pltpu.sync_copy(
            acc_sh.at[pl.ds(sid * NPS, NPS)], out_hbm.at[cid, pl.ds(sid * NPS, NPS)]
        )

    return scatter_k


def _sc_scatter(m, sidx, zero_a):
    return _make_sc_scatter(m.shape[1] * 8)(m, sidx, zero_a)


def _proj_body(h_ref, w1at_ref, w1bt_ref, b1_ref, pq_ref):
    hblk = h_ref[...]
    p = jnp.dot(hblk, w1at_ref[...], preferred_element_type=jnp.float32)
    q = jnp.dot(hblk, w1bt_ref[...], preferred_element_type=jnp.float32)
    pq_ref[...] = jnp.concatenate([p, q + b1_ref[...]], axis=1)


def _tc_proj(h, w1at, w1bt, b1r):
    return pl.pallas_call(
        _proj_body,
        grid=(N // _BN,),
        in_specs=[
            pl.BlockSpec((_BN, D), lambda i: (i, 0)),
            pl.BlockSpec((D, H), lambda i: (0, 0)),
            pl.BlockSpec((D, H), lambda i: (0, 0)),
            pl.BlockSpec((1, H), lambda i: (0, 0)),
        ],
        out_specs=pl.BlockSpec((_BN, D), lambda i: (i, 0)),
        out_shape=jax.ShapeDtypeStruct((N, D), jnp.float32),
    )(h, w1at, w1bt, b1r)


def _mlp_body(ruv_ref, w2_ref, b2_ref, w3_ref, b3_ref, m_ref):
    ru = ruv_ref[0]
    rv = ruv_ref[1]
    s = jnp.concatenate([ru[:, :H] + rv[:, H:], rv[:, :H] + ru[:, H:]], axis=0)
    m1 = jnp.maximum(s, 0.0)
    m2 = jnp.dot(m1, w2_ref[...], preferred_element_type=jnp.float32) + b2_ref[...]
    m2 = jnp.maximum(m2, 0.0)
    m3 = jnp.dot(m2, w3_ref[...], preferred_element_type=jnp.float32) + b3_ref[...]
    pk = _BM // 8
    for r in range(2):
        mr = m3[r * _BM : (r + 1) * _BM]
        packed = jnp.concatenate(
            [mr[k * pk : (k + 1) * pk] for k in range(8)], axis=1
        )
        m_ref[r] = packed


def _tc_mlp(ruv, w2t, b2r, w3t, b3r):
    ne = ruv.shape[1]
    return pl.pallas_call(
        _mlp_body,
        grid=(ne // _BM,),
        in_specs=[
            pl.BlockSpec((2, _BM, D), lambda i: (0, i, 0)),
            pl.BlockSpec((H, H), lambda i: (0, 0)),
            pl.BlockSpec((1, H), lambda i: (0, 0)),
            pl.BlockSpec((H, DE), lambda i: (0, 0)),
            pl.BlockSpec((1, DE), lambda i: (0, 0)),
        ],
        out_specs=pl.BlockSpec((2, _BM // 8, 128), lambda i: (0, i, 0)),
        out_shape=jax.ShapeDtypeStruct((2, ne // 8, 128), jnp.float32),
    )(ruv, w2t, b2r, w3t, b3r)


def _gru_body(
    ap_ref, bp_ref, cp_ref, h_ref, wiht_ref, whht_ref, bih_ref, bhh_ref, ho_ref
):
    a = (ap_ref[0] + ap_ref[1]) + (bp_ref[0] + bp_ref[1]) + (cp_ref[0] + cp_ref[1])
    hblk = h_ref[...]
    gi = jnp.dot(a, wiht_ref[...], preferred_element_type=jnp.float32) + bih_ref[...]
    gh = jnp.dot(hblk, whht_ref[...], preferred_element_type=jnp.float32) + bhh_ref[...]
    r = jax.nn.sigmoid(gi[:, :D] + gh[:, :D])
    z = jax.nn.sigmoid(gi[:, D : 2 * D] + gh[:, D : 2 * D])
    n = jnp.tanh(gi[:, 2 * D :] + r * gh[:, 2 * D :])
    ho_ref[...] = (1.0 - z) * n + z * hblk


def _tc_gru(partsA, partsB, partsC, h, wiht, whht, bihr, bhhr):
    return pl.pallas_call(
        _gru_body,
        grid=(N // _BN,),
        in_specs=[
            pl.BlockSpec((2, _BN, DE), lambda i: (0, i, 0)),
            pl.BlockSpec((2, _BN, DE), lambda i: (0, i, 0)),
            pl.BlockSpec((2, _BN, DE), lambda i: (0, i, 0)),
            pl.BlockSpec((_BN, D), lambda i: (i, 0)),
            pl.BlockSpec((DE, 3 * D), lambda i: (0, 0)),
            pl.BlockSpec((D, 3 * D), lambda i: (0, 0)),
            pl.BlockSpec((1, 3 * D), lambda i: (0, 0)),
            pl.BlockSpec((1, 3 * D), lambda i: (0, 0)),
        ],
        out_specs=pl.BlockSpec((_BN, D), lambda i: (i, 0)),
        out_shape=jax.ShapeDtypeStruct((N, D), jnp.float32),
    )(partsA, partsB, partsC, h, wiht, whht, bihr, bhhr)


def kernel(node_features, edges, W1, b1, W2, b2, W3, b3, W_ih, W_hh, b_ih, b_hh):
    parts_ne = (112000, 112000, 96000)
    eidxs, sidxs = [], []
    off = 0
    for ne in parts_ne:
        edg = edges[:, off : off + ne]
        off += ne
        eidxs.append(edg.reshape(2, NW, ne // NW // CH, CH).transpose(1, 0, 2, 3))
        sidxs.append(
            edg[:, _scatter_perm(ne)].reshape(2, ne // CHS, CHS).transpose(1, 0, 2)
        )
    w1at = W1[:, :D].T
    w1bt = W1[:, D:].T
    b1r = b1.reshape(1, H)
    w2t = W2.T
    b2r = b2.reshape(1, H)
    w3t = W3.T
    b3r = b3.reshape(1, DE)
    wiht = W_ih.T
    whht = W_hh.T
    bihr = b_ih.reshape(1, 3 * D)
    bhhr = b_hh.reshape(1, 3 * D)
    zero_a = jnp.zeros((N, DE), jnp.float32)

    h = node_features
    for _ in range(ITERS):
        pq = _tc_proj(h, w1at, w1bt, b1r)
        ruvs = [_sc_gather(pq, eidx) for eidx in eidxs]
        ms = [_tc_mlp(ruv, w2t, b2r, w3t, b3r) for ruv in ruvs]
        ps = [_sc_scatter(m, sidx, zero_a) for m, sidx in zip(ms, sidxs)]
        h = _tc_gru(ps[0], ps[1], ps[2], h, wiht, whht, bihr, bhhr)
    return h
